# baseline (speedup 1.0000x reference)
import functools

import jax
import jax.numpy as jnp
from jax import lax
from jax.experimental import pallas as pl
from jax.experimental.pallas import tpu as pltpu
from jax.experimental.pallas import tpu_sc as plsc

_N = 10000
_E = 320000
_D = 128
_H1 = 128
_H2 = 64
_OUT = 40

_NC = 2
_NS = 16
_NW = _NC * _NS

_CH = 125
_ROWS = _E // (_NW * _CH)
_NP = 10240
_RPT = _NP // _NS
_DEGW = 8

_MESH = plsc.VectorSubcoreMesh(
    core_axis_name="c", subcore_axis_name="s", num_cores=_NC, num_subcores=_NS)


def _deg_body(dst2, ones, zeros, out, didx, onesb, acc, gsem):
    cid = lax.axis_index("c")
    sid = lax.axis_index("s")
    wid = cid * _NS + sid
    r0 = sid * _RPT
    pltpu.sync_copy(zeros, acc.at[pl.ds(r0, _RPT)])
    pltpu.sync_copy(ones, onesb)
    pltpu.sync_copy(dst2.at[pl.ds(wid * _ROWS, _ROWS)], didx)
    plsc.subcore_barrier()

    def body(i, carry):
        pltpu.sync_copy(onesb, acc.at[didx.at[i]], add=True)
        return carry

    lax.fori_loop(0, _ROWS, body, 0)
    plsc.subcore_barrier()
    pltpu.sync_copy(acc.at[pl.ds(r0, _RPT)], out.at[pl.ds(cid * _NP + r0, _RPT)])


_deg_call = functools.partial(
    pl.kernel,
    out_type=jax.ShapeDtypeStruct((2 * _NP, _DEGW), jnp.float32),
    mesh=_MESH,
    compiler_params=pltpu.CompilerParams(use_tc_tiling_on_sc=False),
    scratch_types=[
        pltpu.VMEM((_ROWS, _CH), jnp.int32),
        pltpu.VMEM((_CH, _DEGW), jnp.float32),
        pltpu.VMEM_SHARED((_NP, _DEGW), jnp.float32),
        pltpu.SemaphoreType.DMA,
    ],
)(_deg_body)


_Q = 16
_NQ = _ROWS // _Q


def _make_agg(width, tiled=True):
    def _agg_body(table, src2, dst2, zeros, out, sidxb, didxb, rows, acc,
                  gsem0, gsem1):
        cid = lax.axis_index("c")
        sid = lax.axis_index("s")
        wid = cid * _NS + sid
        r0 = sid * _RPT
        base = wid * _ROWS
        pltpu.sync_copy(zeros, acc.at[pl.ds(r0, _RPT)])
        pltpu.sync_copy(src2.at[pl.ds(base, _Q)], sidxb.at[0])
        pltpu.sync_copy(dst2.at[pl.ds(base, _Q)], didxb.at[0])
        plsc.subcore_barrier()

        def sidx(c):
            return sidxb.at[(c // _Q) % 2, c % _Q]

        def didx(c):
            return didxb.at[(c // _Q) % 2, c % _Q]

        pltpu.async_copy(table.at[sidx(0)], rows.at[0], gsem0)

        def body(j, carry):
            q = j // (_Q // 2)

            @pl.when((j % (_Q // 2) == 0) & (q < _NQ - 1))
            def _():
                nb = (q + 1) % 2
                off = base + (q + 1) * _Q
                pltpu.sync_copy(src2.at[pl.ds(off, _Q)], sidxb.at[nb])
                pltpu.sync_copy(dst2.at[pl.ds(off, _Q)], didxb.at[nb])

            pltpu.make_async_copy(
                table.at[sidx(0)], rows.at[0], gsem0).wait()
            pltpu.async_copy(table.at[sidx(2 * j + 1)], rows.at[1], gsem1)
            pltpu.sync_copy(rows.at[0], acc.at[didx(2 * j)], add=True)
            pltpu.make_async_copy(
                table.at[sidx(0)], rows.at[1], gsem1).wait()

            @pl.when(j < _ROWS // 2 - 1)
            def _():
                pltpu.async_copy(table.at[sidx(2 * j + 2)], rows.at[0], gsem0)

            pltpu.sync_copy(rows.at[1], acc.at[didx(2 * j + 1)], add=True)
            return carry

        lax.fori_loop(0, _ROWS // 2, body, 0)
        plsc.subcore_barrier()
        pltpu.sync_copy(acc.at[pl.ds(r0, _RPT)],
                        out.at[pl.ds(cid * _NP + r0, _RPT)])

    return functools.partial(
        pl.kernel,
        out_type=jax.ShapeDtypeStruct((2 * _NP, width), jnp.float32),
        mesh=_MESH,
        compiler_params=None if tiled else pltpu.CompilerParams(
            use_tc_tiling_on_sc=False),
        scratch_types=[
            pltpu.VMEM((2, _Q, _CH), jnp.int32),
            pltpu.VMEM((2, _Q, _CH), jnp.int32),
            pltpu.VMEM((2, _CH, width), jnp.float32),
            pltpu.VMEM_SHARED((_NP, width), jnp.float32),
            pltpu.SemaphoreType.DMA,
            pltpu.SemaphoreType.DMA,
        ],
    )(_agg_body)


_agg128 = _make_agg(_D)
_agg64 = _make_agg(_H2, tiled=False)


_BLK = 80
_GRID = _N // _BLK
_POFF = _NP // _BLK


def _se_body(x_ref, uw, ub, f1w, f1b, f2w, f2b, w1, d0, d1,
             w_out, t1_out, dinv_out):
    x = x_ref[...]
    h = jnp.dot(x, uw[...], preferred_element_type=jnp.float32) + ub[...]
    t = jnp.maximum(
        jnp.dot(h, f1w[...], preferred_element_type=jnp.float32) + f1b[...], 0.0)
    wgt = jax.nn.sigmoid(
        jnp.dot(t, f2w[...], preferred_element_type=jnp.float32) + f2b[...])
    deg = d0[...][:, 0:1] + d1[...][:, 0:1] + 1.0
    dinv = lax.rsqrt(deg)
    w_out[...] = wgt
    t1_out[...] = jnp.dot(h * wgt * dinv, w1[...],
                          preferred_element_type=jnp.float32)
    dinv_out[...] = dinv


def _mid_body(a0, a1, t1, dinv_ref, b1, w2, t2_out):
    dinv = dinv_ref[...]
    h1 = jnp.maximum(dinv * (a0[...] + a1[...] + t1[...]) + b1[...], 0.0)
    t2_out[...] = dinv * jnp.dot(h1, w2[...],
                                 preferred_element_type=jnp.float32)


def _out_body(a0, a1, t2, dinv_ref, b2, fcw, fcb, lp_out):
    dinv = dinv_ref[...]
    h2 = jnp.maximum(dinv * (a0[...] + a1[...] + t2[...]) + b2[...], 0.0)
    logits = jnp.dot(h2, fcw[...], preferred_element_type=jnp.float32) + fcb[...]
    m = jnp.max(logits, axis=1, keepdims=True)
    s = logits - m
    lp_out[...] = s - jnp.log(jnp.sum(jnp.exp(s), axis=1, keepdims=True))


def _row_spec(w):
    return pl.BlockSpec((_BLK, w), lambda i: (i, 0))


def _p1_spec(w):
    return pl.BlockSpec((_BLK, w), lambda i: (i + _POFF, 0))


def _full_spec(r, c):
    return pl.BlockSpec((r, c), lambda i: (0, 0))


_se_call = pl.pallas_call(
    _se_body,
    grid=(_GRID,),
    in_specs=[
        _row_spec(_D), _full_spec(_D, _D), _full_spec(1, _D),
        _full_spec(_D, _D // 4), _full_spec(1, _D // 4),
        _full_spec(_D // 4, _D), _full_spec(1, _D),
        _full_spec(_D, _H1), _row_spec(_DEGW), _p1_spec(_DEGW),
    ],
    out_specs=[_row_spec(_D), _row_spec(_H1), _row_spec(1)],
    out_shape=[
        jax.ShapeDtypeStruct((_N, _D), jnp.float32),
        jax.ShapeDtypeStruct((_N, _H1), jnp.float32),
        jax.ShapeDtypeStruct((_N, 1), jnp.float32),
    ],
)

_mid_call = pl.pallas_call(
    _mid_body,
    grid=(_GRID,),
    in_specs=[
        _row_spec(_H1), _p1_spec(_H1), _row_spec(_H1), _row_spec(1),
        _full_spec(1, _H1), _full_spec(_H1, _H2),
    ],
    out_specs=[_row_spec(_H2)],
    out_shape=[jax.ShapeDtypeStruct((_N, _H2), jnp.float32)],
)

_out_call = pl.pallas_call(
    _out_body,
    grid=(_GRID,),
    in_specs=[
        _row_spec(_H2), _p1_spec(_H2), _row_spec(_H2), _row_spec(1),
        _full_spec(1, _H2), _full_spec(_H2, _OUT), _full_spec(1, _OUT),
    ],
    out_specs=[_row_spec(_OUT)],
    out_shape=[jax.ShapeDtypeStruct((_N, _OUT), jnp.float32)],
)


def kernel(x, edge_index, U_W, U_b, fc1_W, fc1_b, fc2_W, fc2_b,
           W1, b1, W2, b2, fc_W, fc_b):
    src2 = edge_index[0].astype(jnp.int32).reshape(_E // _CH, _CH)
    dst2 = edge_index[1].astype(jnp.int32).reshape(_E // _CH, _CH)

    ones_p = jnp.ones((_CH, _DEGW), jnp.float32)
    zeros_deg = jnp.zeros((_RPT, _DEGW), jnp.float32)
    zeros128 = jnp.zeros((_RPT, _D), jnp.float32)
    zeros64 = jnp.zeros((_RPT, _H2), jnp.float32)

    degp = _deg_call(dst2, ones_p, zeros_deg)
    w, t1, dinv = _se_call(
        x, U_W, U_b.reshape(1, _D), fc1_W, fc1_b.reshape(1, _D // 4),
        fc2_W, fc2_b.reshape(1, _D), W1, degp, degp)
    agg1 = _agg128(t1, src2, dst2, zeros128)
    (t2,) = _mid_call(agg1, agg1, t1, dinv, b1.reshape(1, _H1), W2)
    agg2 = _agg64(t2, src2, dst2, zeros64)
    (logp,) = _out_call(agg2, agg2, t2, dinv,
                        b2.reshape(1, _H2), fc_W, fc_b.reshape(1, _OUT))
    return (logp, w)


# speedup vs baseline: 1.4100x; 1.4100x over previous
import functools

import jax
import jax.numpy as jnp
from jax import lax
from jax.experimental import pallas as pl
from jax.experimental.pallas import tpu as pltpu
from jax.experimental.pallas import tpu_sc as plsc

_N = 10000
_E = 320000
_D = 128
_H1 = 128
_H2 = 64
_OUT = 40

_NC = 2
_NS = 16
_NW = _NC * _NS

_CH = 125
_ROWS = _E // (_NW * _CH)
_NP = 10240
_RPT = _NP // _NS
_DEGW = 8

_MESH = plsc.VectorSubcoreMesh(
    core_axis_name="c", subcore_axis_name="s", num_cores=_NC, num_subcores=_NS)


def _deg_body(dst2, ones, zeros, out, didx, onesb, acc, gsem):
    cid = lax.axis_index("c")
    sid = lax.axis_index("s")
    wid = cid * _NS + sid
    r0 = sid * _RPT
    pltpu.sync_copy(zeros, acc.at[pl.ds(r0, _RPT)])
    pltpu.sync_copy(ones, onesb)
    pltpu.sync_copy(dst2.at[pl.ds(wid * _ROWS, _ROWS)], didx)
    plsc.subcore_barrier()

    def body(i, carry):
        pltpu.sync_copy(onesb, acc.at[didx.at[i]], add=True)
        return carry

    lax.fori_loop(0, _ROWS, body, 0)
    plsc.subcore_barrier()
    pltpu.sync_copy(acc.at[pl.ds(r0, _RPT)], out.at[pl.ds(cid * _NP + r0, _RPT)])


_deg_call = functools.partial(
    pl.kernel,
    out_type=jax.ShapeDtypeStruct((2 * _NP, _DEGW), jnp.float32),
    mesh=_MESH,
    compiler_params=pltpu.CompilerParams(use_tc_tiling_on_sc=False),
    scratch_types=[
        pltpu.VMEM((_ROWS, _CH), jnp.int32),
        pltpu.VMEM((_CH, _DEGW), jnp.float32),
        pltpu.VMEM_SHARED((_NP, _DEGW), jnp.float32),
        pltpu.SemaphoreType.DMA,
    ],
)(_deg_body)


_Q = 16
_NQ = _ROWS // _Q


def _make_agg(width, tiled=True):
    def _agg_body(table, src2, dst2, zeros, out, sidxb, didxb, rows, acc,
                  gsem0, gsem1):
        cid = lax.axis_index("c")
        sid = lax.axis_index("s")
        wid = cid * _NS + sid
        r0 = sid * _RPT
        base = wid * _ROWS
        pltpu.sync_copy(zeros, acc.at[pl.ds(r0, _RPT)])
        pltpu.sync_copy(src2.at[pl.ds(base, _Q)], sidxb.at[0])
        pltpu.sync_copy(dst2.at[pl.ds(base, _Q)], didxb.at[0])
        plsc.subcore_barrier()

        def sidx(c):
            return sidxb.at[(c // _Q) % 2, c % _Q]

        def didx(c):
            return didxb.at[(c // _Q) % 2, c % _Q]

        pltpu.async_copy(table.at[sidx(0)], rows.at[0], gsem0)

        def body(j, carry):
            q = j // (_Q // 2)

            @pl.when((j % (_Q // 2) == 0) & (q < _NQ - 1))
            def _():
                nb = (q + 1) % 2
                off = base + (q + 1) * _Q
                pltpu.sync_copy(src2.at[pl.ds(off, _Q)], sidxb.at[nb])
                pltpu.sync_copy(dst2.at[pl.ds(off, _Q)], didxb.at[nb])

            pltpu.make_async_copy(
                table.at[sidx(0)], rows.at[0], gsem0).wait()
            pltpu.async_copy(table.at[sidx(2 * j + 1)], rows.at[1], gsem1)
            pltpu.sync_copy(rows.at[0], acc.at[didx(2 * j)], add=True)
            pltpu.make_async_copy(
                table.at[sidx(0)], rows.at[1], gsem1).wait()

            @pl.when(j < _ROWS // 2 - 1)
            def _():
                pltpu.async_copy(table.at[sidx(2 * j + 2)], rows.at[0], gsem0)

            pltpu.sync_copy(rows.at[1], acc.at[didx(2 * j + 1)], add=True)
            return carry

        lax.fori_loop(0, _ROWS // 2, body, 0)
        plsc.subcore_barrier()
        pltpu.sync_copy(acc.at[pl.ds(r0, _RPT)],
                        out.at[pl.ds(cid * _NP + r0, _RPT)])

    return functools.partial(
        pl.kernel,
        out_type=jax.ShapeDtypeStruct((2 * _NP, width), jnp.float32),
        mesh=_MESH,
        compiler_params=None if tiled else pltpu.CompilerParams(
            use_tc_tiling_on_sc=False),
        scratch_types=[
            pltpu.VMEM((2, _Q, _CH), jnp.int32),
            pltpu.VMEM((2, _Q, _CH), jnp.int32),
            pltpu.VMEM((2, _CH, width), jnp.float32),
            pltpu.VMEM_SHARED((_NP, width), jnp.float32),
            pltpu.SemaphoreType.DMA,
            pltpu.SemaphoreType.DMA,
        ],
    )(_agg_body)


_agg128 = _make_agg(_D)
_agg64 = _make_agg(_H2, tiled=False)


_BLK = 400
_GRID = _N // _BLK


def _se_body(x_ref, uw, ub, f1w, f1b, f2w, f2b, w1, d0, d1,
             w_out, t1_out, dinv_out):
    x = x_ref[...]
    h = jnp.dot(x, uw[...], preferred_element_type=jnp.float32) + ub[...]
    t = jnp.maximum(
        jnp.dot(h, f1w[...], preferred_element_type=jnp.float32) + f1b[...], 0.0)
    wgt = jax.nn.sigmoid(
        jnp.dot(t, f2w[...], preferred_element_type=jnp.float32) + f2b[...])
    deg = d0[...][:, 0:1] + d1[...][:, 0:1] + 1.0
    dinv = lax.rsqrt(deg)
    w_out[...] = wgt
    t1_out[...] = jnp.dot(h * wgt * dinv, w1[...],
                          preferred_element_type=jnp.float32)
    dinv_out[...] = dinv


def _mid_body(a0, a1, t1, dinv_ref, b1, w2, t2_out):
    dinv = dinv_ref[...]
    h1 = jnp.maximum(dinv * (a0[...] + a1[...] + t1[...]) + b1[...], 0.0)
    t2_out[...] = dinv * jnp.dot(h1, w2[...],
                                 preferred_element_type=jnp.float32)


def _out_body(a0, a1, t2, dinv_ref, b2, fcw, fcb, lp_out):
    dinv = dinv_ref[...]
    h2 = jnp.maximum(dinv * (a0[...] + a1[...] + t2[...]) + b2[...], 0.0)
    logits = jnp.dot(h2, fcw[...], preferred_element_type=jnp.float32) + fcb[...]
    m = jnp.max(logits, axis=1, keepdims=True)
    s = logits - m
    lp_out[...] = s - jnp.log(jnp.sum(jnp.exp(s), axis=1, keepdims=True))


def _row_spec(w):
    return pl.BlockSpec((_BLK, w), lambda i: (i, 0))


def _full_spec(r, c):
    return pl.BlockSpec((r, c), lambda i: (0, 0))


_se_call = pl.pallas_call(
    _se_body,
    grid=(_GRID,),
    in_specs=[
        _row_spec(_D), _full_spec(_D, _D), _full_spec(1, _D),
        _full_spec(_D, _D // 4), _full_spec(1, _D // 4),
        _full_spec(_D // 4, _D), _full_spec(1, _D),
        _full_spec(_D, _H1), _row_spec(_DEGW), _row_spec(_DEGW),
    ],
    out_specs=[_row_spec(_D), _row_spec(_H1), _row_spec(1)],
    out_shape=[
        jax.ShapeDtypeStruct((_N, _D), jnp.float32),
        jax.ShapeDtypeStruct((_N, _H1), jnp.float32),
        jax.ShapeDtypeStruct((_N, 1), jnp.float32),
    ],
)

_mid_call = pl.pallas_call(
    _mid_body,
    grid=(_GRID,),
    in_specs=[
        _row_spec(_H1), _row_spec(_H1), _row_spec(_H1), _row_spec(1),
        _full_spec(1, _H1), _full_spec(_H1, _H2),
    ],
    out_specs=[_row_spec(_H2)],
    out_shape=[jax.ShapeDtypeStruct((_N, _H2), jnp.float32)],
)

_out_call = pl.pallas_call(
    _out_body,
    grid=(_GRID,),
    in_specs=[
        _row_spec(_H2), _row_spec(_H2), _row_spec(_H2), _row_spec(1),
        _full_spec(1, _H2), _full_spec(_H2, _OUT), _full_spec(1, _OUT),
    ],
    out_specs=[_row_spec(_OUT)],
    out_shape=[jax.ShapeDtypeStruct((_N, _OUT), jnp.float32)],
)


def kernel(x, edge_index, U_W, U_b, fc1_W, fc1_b, fc2_W, fc2_b,
           W1, b1, W2, b2, fc_W, fc_b):
    src2 = edge_index[0].astype(jnp.int32).reshape(_E // _CH, _CH)
    dst2 = edge_index[1].astype(jnp.int32).reshape(_E // _CH, _CH)

    ones_p = jnp.ones((_CH, _DEGW), jnp.float32)
    zeros_deg = jnp.zeros((_RPT, _DEGW), jnp.float32)
    zeros128 = jnp.zeros((_RPT, _D), jnp.float32)
    zeros64 = jnp.zeros((_RPT, _H2), jnp.float32)

    degp = _deg_call(dst2, ones_p, zeros_deg)
    w, t1, dinv = _se_call(
        x, U_W, U_b.reshape(1, _D), fc1_W, fc1_b.reshape(1, _D // 4),
        fc2_W, fc2_b.reshape(1, _D), W1, degp[:_N], degp[_NP:_NP + _N])
    agg1 = _agg128(t1, src2, dst2, zeros128)
    (t2,) = _mid_call(agg1[:_N], agg1[_NP:_NP + _N], t1, dinv,
                      b1.reshape(1, _H1), W2)
    agg2 = _agg64(t2, src2, dst2, zeros64)
    (logp,) = _out_call(agg2[:_N], agg2[_NP:_NP + _N], t2, dinv,
                        b2.reshape(1, _H2), fc_W, fc_b.reshape(1, _OUT))
    return (logp, w)


# speedup vs baseline: 1.4153x; 1.0038x over previous
import functools

import jax
import jax.numpy as jnp
from jax import lax
from jax.experimental import pallas as pl
from jax.experimental.pallas import tpu as pltpu
from jax.experimental.pallas import tpu_sc as plsc

_N = 10000
_E = 320000
_D = 128
_H1 = 128
_H2 = 64
_OUT = 40

_NC = 2
_NS = 16
_NW = _NC * _NS

_CH = 125
_ROWS = _E // (_NW * _CH)
_NP = 10240
_RPT = _NP // _NS
_DEGW = 8

_MESH = plsc.VectorSubcoreMesh(
    core_axis_name="c", subcore_axis_name="s", num_cores=_NC, num_subcores=_NS)


def _deg_body(dst2, ones, zeros, out, didx, onesb, acc, gsem):
    cid = lax.axis_index("c")
    sid = lax.axis_index("s")
    wid = cid * _NS + sid
    r0 = sid * _RPT
    pltpu.sync_copy(zeros, acc.at[pl.ds(r0, _RPT)])
    pltpu.sync_copy(ones, onesb)
    pltpu.sync_copy(dst2.at[pl.ds(wid * _ROWS, _ROWS)], didx)
    plsc.subcore_barrier()

    def body(i, carry):
        pltpu.sync_copy(onesb, acc.at[didx.at[i]], add=True)
        return carry

    lax.fori_loop(0, _ROWS, body, 0)
    plsc.subcore_barrier()
    pltpu.sync_copy(acc.at[pl.ds(r0, _RPT)], out.at[pl.ds(cid * _NP + r0, _RPT)])


_deg_call = functools.partial(
    pl.kernel,
    out_type=jax.ShapeDtypeStruct((2 * _NP, _DEGW), jnp.float32),
    mesh=_MESH,
    compiler_params=pltpu.CompilerParams(use_tc_tiling_on_sc=False),
    scratch_types=[
        pltpu.VMEM((_ROWS, _CH), jnp.int32),
        pltpu.VMEM((_CH, _DEGW), jnp.float32),
        pltpu.VMEM_SHARED((_NP, _DEGW), jnp.float32),
        pltpu.SemaphoreType.DMA,
    ],
)(_deg_body)


_Q = 16
_NQ = _ROWS // _Q


def _make_agg(width, tiled=True):
    def _agg_body(table, src2, dst2, zeros, out, sidxb, didxb, rows, acc,
                  gsem0, gsem1, ssem0, ssem1):
        cid = lax.axis_index("c")
        sid = lax.axis_index("s")
        wid = cid * _NS + sid
        r0 = sid * _RPT
        base = wid * _ROWS
        pltpu.sync_copy(zeros, acc.at[pl.ds(r0, _RPT)])
        pltpu.sync_copy(src2.at[pl.ds(base, _Q)], sidxb.at[0])
        pltpu.sync_copy(dst2.at[pl.ds(base, _Q)], didxb.at[0])
        plsc.subcore_barrier()

        def sidx(c):
            return sidxb.at[(c // _Q) % 2, c % _Q]

        def didx(c):
            return didxb.at[(c // _Q) % 2, c % _Q]

        gsem = (gsem0, gsem1)
        ssem = (ssem0, ssem1)

        def wait_g(b):
            pltpu.make_async_copy(
                table.at[sidx(0)], rows.at[b], gsem[b]).wait()

        def wait_s(b):
            pltpu.make_async_copy(
                rows.at[b], acc.at[didx(0)], ssem[b]).wait()

        def issue_g(c, b):
            pltpu.async_copy(table.at[sidx(c)], rows.at[b], gsem[b])

        def issue_s(c, b):
            pltpu.async_copy(rows.at[b], acc.at[didx(c)], ssem[b], add=True)

        issue_g(0, 0)
        wait_g(0)
        issue_s(0, 0)
        issue_g(1, 1)

        def body(j, carry):
            qn = (2 * j + 2) // _Q

            @pl.when((j % (_Q // 2) == _Q // 2 - 1) & (qn < _NQ))
            def _():
                off = base + qn * _Q
                pltpu.sync_copy(src2.at[pl.ds(off, _Q)], sidxb.at[qn % 2])
                pltpu.sync_copy(dst2.at[pl.ds(off, _Q)], didxb.at[qn % 2])

            wait_g(1)
            issue_s(2 * j + 1, 1)
            wait_s(0)
            issue_g(2 * j + 2, 0)
            wait_g(0)
            issue_s(2 * j + 2, 0)
            wait_s(1)

            @pl.when(j < _ROWS // 2 - 2)
            def _():
                issue_g(2 * j + 3, 1)

            return carry

        lax.fori_loop(0, _ROWS // 2 - 1, body, 0)
        issue_g(_ROWS - 1, 1)
        wait_g(1)
        issue_s(_ROWS - 1, 1)
        wait_s(0)
        wait_s(1)
        plsc.subcore_barrier()
        pltpu.sync_copy(acc.at[pl.ds(r0, _RPT)],
                        out.at[pl.ds(cid * _NP + r0, _RPT)])

    return functools.partial(
        pl.kernel,
        out_type=jax.ShapeDtypeStruct((2 * _NP, width), jnp.float32),
        mesh=_MESH,
        compiler_params=None if tiled else pltpu.CompilerParams(
            use_tc_tiling_on_sc=False),
        scratch_types=[
            pltpu.VMEM((2, _Q, _CH), jnp.int32),
            pltpu.VMEM((2, _Q, _CH), jnp.int32),
            pltpu.VMEM((2, _CH, width), jnp.float32),
            pltpu.VMEM_SHARED((_NP, width), jnp.float32),
            pltpu.SemaphoreType.DMA,
            pltpu.SemaphoreType.DMA,
            pltpu.SemaphoreType.DMA,
            pltpu.SemaphoreType.DMA,
        ],
    )(_agg_body)


_agg128 = _make_agg(_D)
_agg64 = _make_agg(_H2, tiled=False)


_BLK = 400
_GRID = _N // _BLK


def _se_body(x_ref, uw, ub, f1w, f1b, f2w, f2b, w1, d0, d1,
             w_out, t1_out, dinv_out):
    x = x_ref[...]
    h = jnp.dot(x, uw[...], preferred_element_type=jnp.float32) + ub[...]
    t = jnp.maximum(
        jnp.dot(h, f1w[...], preferred_element_type=jnp.float32) + f1b[...], 0.0)
    wgt = jax.nn.sigmoid(
        jnp.dot(t, f2w[...], preferred_element_type=jnp.float32) + f2b[...])
    deg = d0[...][:, 0:1] + d1[...][:, 0:1] + 1.0
    dinv = lax.rsqrt(deg)
    w_out[...] = wgt
    t1_out[...] = jnp.dot(h * wgt * dinv, w1[...],
                          preferred_element_type=jnp.float32)
    dinv_out[...] = dinv


def _mid_body(a0, a1, t1, dinv_ref, b1, w2, t2_out):
    dinv = dinv_ref[...]
    h1 = jnp.maximum(dinv * (a0[...] + a1[...] + t1[...]) + b1[...], 0.0)
    t2_out[...] = dinv * jnp.dot(h1, w2[...],
                                 preferred_element_type=jnp.float32)


def _out_body(a0, a1, t2, dinv_ref, b2, fcw, fcb, lp_out):
    dinv = dinv_ref[...]
    h2 = jnp.maximum(dinv * (a0[...] + a1[...] + t2[...]) + b2[...], 0.0)
    logits = jnp.dot(h2, fcw[...], preferred_element_type=jnp.float32) + fcb[...]
    m = jnp.max(logits, axis=1, keepdims=True)
    s = logits - m
    lp_out[...] = s - jnp.log(jnp.sum(jnp.exp(s), axis=1, keepdims=True))


def _row_spec(w):
    return pl.BlockSpec((_BLK, w), lambda i: (i, 0))


def _full_spec(r, c):
    return pl.BlockSpec((r, c), lambda i: (0, 0))


_se_call = pl.pallas_call(
    _se_body,
    grid=(_GRID,),
    in_specs=[
        _row_spec(_D), _full_spec(_D, _D), _full_spec(1, _D),
        _full_spec(_D, _D // 4), _full_spec(1, _D // 4),
        _full_spec(_D // 4, _D), _full_spec(1, _D),
        _full_spec(_D, _H1), _row_spec(_DEGW), _row_spec(_DEGW),
    ],
    out_specs=[_row_spec(_D), _row_spec(_H1), _row_spec(1)],
    out_shape=[
        jax.ShapeDtypeStruct((_N, _D), jnp.float32),
        jax.ShapeDtypeStruct((_N, _H1), jnp.float32),
        jax.ShapeDtypeStruct((_N, 1), jnp.float32),
    ],
)

_mid_call = pl.pallas_call(
    _mid_body,
    grid=(_GRID,),
    in_specs=[
        _row_spec(_H1), _row_spec(_H1), _row_spec(_H1), _row_spec(1),
        _full_spec(1, _H1), _full_spec(_H1, _H2),
    ],
    out_specs=[_row_spec(_H2)],
    out_shape=[jax.ShapeDtypeStruct((_N, _H2), jnp.float32)],
)

_out_call = pl.pallas_call(
    _out_body,
    grid=(_GRID,),
    in_specs=[
        _row_spec(_H2), _row_spec(_H2), _row_spec(_H2), _row_spec(1),
        _full_spec(1, _H2), _full_spec(_H2, _OUT), _full_spec(1, _OUT),
    ],
    out_specs=[_row_spec(_OUT)],
    out_shape=[jax.ShapeDtypeStruct((_N, _OUT), jnp.float32)],
)


def kernel(x, edge_index, U_W, U_b, fc1_W, fc1_b, fc2_W, fc2_b,
           W1, b1, W2, b2, fc_W, fc_b):
    src2 = edge_index[0].astype(jnp.int32).reshape(_E // _CH, _CH)
    dst2 = edge_index[1].astype(jnp.int32).reshape(_E // _CH, _CH)

    ones_p = jnp.ones((_CH, _DEGW), jnp.float32)
    zeros_deg = jnp.zeros((_RPT, _DEGW), jnp.float32)
    zeros128 = jnp.zeros((_RPT, _D), jnp.float32)
    zeros64 = jnp.zeros((_RPT, _H2), jnp.float32)

    degp = _deg_call(dst2, ones_p, zeros_deg)
    w, t1, dinv = _se_call(
        x, U_W, U_b.reshape(1, _D), fc1_W, fc1_b.reshape(1, _D // 4),
        fc2_W, fc2_b.reshape(1, _D), W1, degp[:_N], degp[_NP:_NP + _N])
    agg1 = _agg128(t1, src2, dst2, zeros128)
    (t2,) = _mid_call(agg1[:_N], agg1[_NP:_NP + _N], t1, dinv,
                      b1.reshape(1, _H1), W2)
    agg2 = _agg64(t2, src2, dst2, zeros64)
    (logp,) = _out_call(agg2[:_N], agg2[_NP:_NP + _N], t2, dinv,
                        b2.reshape(1, _H2), fc_W, fc_b.reshape(1, _OUT))
    return (logp, w)


# speedup vs baseline: 1.5194x; 1.0735x over previous
import functools

import jax
import jax.numpy as jnp
from jax import lax
from jax.experimental import pallas as pl
from jax.experimental.pallas import tpu as pltpu
from jax.experimental.pallas import tpu_sc as plsc

_N = 10000
_E = 320000
_D = 128
_H1 = 128
_H2 = 64
_OUT = 40

_NC = 2
_NS = 16
_NW = _NC * _NS

_CH = 125
_ROWS = _E // (_NW * _CH)
_NP = 10240
_RPT = _NP // _NS
_DEGW = 8

_MESH = plsc.VectorSubcoreMesh(
    core_axis_name="c", subcore_axis_name="s", num_cores=_NC, num_subcores=_NS)


def _deg_body(dst2, ones, zeros, out, didx, onesb, acc, gsem):
    cid = lax.axis_index("c")
    sid = lax.axis_index("s")
    wid = cid * _NS + sid
    r0 = sid * _RPT
    pltpu.sync_copy(zeros, acc.at[pl.ds(r0, _RPT)])
    pltpu.sync_copy(ones, onesb)
    pltpu.sync_copy(dst2.at[pl.ds(wid * _ROWS, _ROWS)], didx)
    plsc.subcore_barrier()

    def body(i, carry):
        pltpu.sync_copy(onesb, acc.at[didx.at[i]], add=True)
        return carry

    lax.fori_loop(0, _ROWS, body, 0)
    plsc.subcore_barrier()
    pltpu.sync_copy(acc.at[pl.ds(r0, _RPT)], out.at[pl.ds(cid * _NP + r0, _RPT)])


_deg_call = functools.partial(
    pl.kernel,
    out_type=jax.ShapeDtypeStruct((2 * _NP, _DEGW), jnp.float32),
    mesh=_MESH,
    compiler_params=pltpu.CompilerParams(use_tc_tiling_on_sc=False),
    scratch_types=[
        pltpu.VMEM((_ROWS, _CH), jnp.int32),
        pltpu.VMEM((_CH, _DEGW), jnp.float32),
        pltpu.VMEM_SHARED((_NP, _DEGW), jnp.float32),
        pltpu.SemaphoreType.DMA,
    ],
)(_deg_body)


_Q = 16
_NQ = _ROWS // _Q


def _make_agg(width, tiled=True):
    def _agg_body(table, src2, dst2, zeros, out, sidxb, didxb, rows, acc,
                  gsem0, gsem1, ssem0, ssem1):
        cid = lax.axis_index("c")
        sid = lax.axis_index("s")
        wid = cid * _NS + sid
        r0 = sid * _RPT
        base = wid * _ROWS
        pltpu.sync_copy(zeros, acc.at[pl.ds(r0, _RPT)])
        pltpu.sync_copy(src2.at[pl.ds(base, _Q)], sidxb.at[0])
        pltpu.sync_copy(dst2.at[pl.ds(base, _Q)], didxb.at[0])
        plsc.subcore_barrier()

        def sidx(c):
            return sidxb.at[(c // _Q) % 2, c % _Q]

        def didx(c):
            return didxb.at[(c // _Q) % 2, c % _Q]

        gsem = (gsem0, gsem1)
        ssem = (ssem0, ssem1)

        def wait_g(b):
            pltpu.make_async_copy(
                table.at[sidx(0)], rows.at[b], gsem[b]).wait()

        def wait_s(b):
            pltpu.make_async_copy(
                rows.at[b], acc.at[didx(0)], ssem[b]).wait()

        def issue_g(c, b):
            pltpu.async_copy(table.at[sidx(c)], rows.at[b], gsem[b])

        def issue_s(c, b):
            pltpu.async_copy(rows.at[b], acc.at[didx(c)], ssem[b], add=True)

        issue_g(0, 0)
        wait_g(0)
        issue_s(0, 0)
        issue_g(1, 1)

        def body(j, carry):
            qn = (2 * j + 2) // _Q

            @pl.when((j % (_Q // 2) == _Q // 2 - 1) & (qn < _NQ))
            def _():
                off = base + qn * _Q
                pltpu.sync_copy(src2.at[pl.ds(off, _Q)], sidxb.at[qn % 2])
                pltpu.sync_copy(dst2.at[pl.ds(off, _Q)], didxb.at[qn % 2])

            wait_g(1)
            issue_s(2 * j + 1, 1)
            wait_s(0)
            issue_g(2 * j + 2, 0)
            wait_g(0)
            issue_s(2 * j + 2, 0)
            wait_s(1)

            @pl.when(j < _ROWS // 2 - 2)
            def _():
                issue_g(2 * j + 3, 1)

            return carry

        lax.fori_loop(0, _ROWS // 2 - 1, body, 0)
        issue_g(_ROWS - 1, 1)
        wait_g(1)
        issue_s(_ROWS - 1, 1)
        wait_s(0)
        wait_s(1)
        plsc.subcore_barrier()
        pltpu.sync_copy(acc.at[pl.ds(r0, _RPT)],
                        out.at[pl.ds(cid * _NP + r0, _RPT)])

    return functools.partial(
        pl.kernel,
        out_type=jax.ShapeDtypeStruct((2 * _NP, width), jnp.float32),
        mesh=_MESH,
        compiler_params=None if tiled else pltpu.CompilerParams(
            use_tc_tiling_on_sc=False),
        scratch_types=[
            pltpu.VMEM((2, _Q, _CH), jnp.int32),
            pltpu.VMEM((2, _Q, _CH), jnp.int32),
            pltpu.VMEM((2, _CH, width), jnp.float32),
            pltpu.VMEM_SHARED((_NP, width), jnp.float32),
            pltpu.SemaphoreType.DMA,
            pltpu.SemaphoreType.DMA,
            pltpu.SemaphoreType.DMA,
            pltpu.SemaphoreType.DMA,
        ],
    )(_agg_body)


_agg128 = _make_agg(_D)
_agg64 = _make_agg(_H2, tiled=False)


_BLK = 1000
_GRID = _N // _BLK


def _se_body(x_ref, uw, ub, f1w, f1b, f2w, f2b, w1, d0, d1,
             w_out, t1_out, dinv_out):
    x = x_ref[...]
    h = jnp.dot(x, uw[...], preferred_element_type=jnp.float32) + ub[...]
    t = jnp.maximum(
        jnp.dot(h, f1w[...], preferred_element_type=jnp.float32) + f1b[...], 0.0)
    wgt = jax.nn.sigmoid(
        jnp.dot(t, f2w[...], preferred_element_type=jnp.float32) + f2b[...])
    deg = d0[...][:, 0:1] + d1[...][:, 0:1] + 1.0
    dinv = lax.rsqrt(deg)
    w_out[...] = wgt
    t1_out[...] = jnp.dot(h * wgt * dinv, w1[...],
                          preferred_element_type=jnp.float32)
    dinv_out[...] = dinv


def _mid_body(a0, a1, t1, dinv_ref, b1, w2, t2_out):
    dinv = dinv_ref[...]
    h1 = jnp.maximum(dinv * (a0[...] + a1[...] + t1[...]) + b1[...], 0.0)
    t2_out[...] = dinv * jnp.dot(h1, w2[...],
                                 preferred_element_type=jnp.float32)


def _out_body(a0, a1, t2, dinv_ref, b2, fcw, fcb, lp_out):
    dinv = dinv_ref[...]
    h2 = jnp.maximum(dinv * (a0[...] + a1[...] + t2[...]) + b2[...], 0.0)
    logits = jnp.dot(h2, fcw[...], preferred_element_type=jnp.float32) + fcb[...]
    m = jnp.max(logits, axis=1, keepdims=True)
    s = logits - m
    lp_out[...] = s - jnp.log(jnp.sum(jnp.exp(s), axis=1, keepdims=True))


def _row_spec(w):
    return pl.BlockSpec((_BLK, w), lambda i: (i, 0))


def _full_spec(r, c):
    return pl.BlockSpec((r, c), lambda i: (0, 0))


_se_call = pl.pallas_call(
    _se_body,
    grid=(_GRID,),
    in_specs=[
        _row_spec(_D), _full_spec(_D, _D), _full_spec(1, _D),
        _full_spec(_D, _D // 4), _full_spec(1, _D // 4),
        _full_spec(_D // 4, _D), _full_spec(1, _D),
        _full_spec(_D, _H1), _row_spec(_DEGW), _row_spec(_DEGW),
    ],
    out_specs=[_row_spec(_D), _row_spec(_H1), _row_spec(1)],
    out_shape=[
        jax.ShapeDtypeStruct((_N, _D), jnp.float32),
        jax.ShapeDtypeStruct((_N, _H1), jnp.float32),
        jax.ShapeDtypeStruct((_N, 1), jnp.float32),
    ],
)

_mid_call = pl.pallas_call(
    _mid_body,
    grid=(_GRID,),
    in_specs=[
        _row_spec(_H1), _row_spec(_H1), _row_spec(_H1), _row_spec(1),
        _full_spec(1, _H1), _full_spec(_H1, _H2),
    ],
    out_specs=[_row_spec(_H2)],
    out_shape=[jax.ShapeDtypeStruct((_N, _H2), jnp.float32)],
)

_out_call = pl.pallas_call(
    _out_body,
    grid=(_GRID,),
    in_specs=[
        _row_spec(_H2), _row_spec(_H2), _row_spec(_H2), _row_spec(1),
        _full_spec(1, _H2), _full_spec(_H2, _OUT), _full_spec(1, _OUT),
    ],
    out_specs=[_row_spec(_OUT)],
    out_shape=[jax.ShapeDtypeStruct((_N, _OUT), jnp.float32)],
)


def kernel(x, edge_index, U_W, U_b, fc1_W, fc1_b, fc2_W, fc2_b,
           W1, b1, W2, b2, fc_W, fc_b):
    src2 = edge_index[0].astype(jnp.int32).reshape(_E // _CH, _CH)
    dst2 = edge_index[1].astype(jnp.int32).reshape(_E // _CH, _CH)

    ones_p = jnp.ones((_CH, _DEGW), jnp.float32)
    zeros_deg = jnp.zeros((_RPT, _DEGW), jnp.float32)
    zeros128 = jnp.zeros((_RPT, _D), jnp.float32)
    zeros64 = jnp.zeros((_RPT, _H2), jnp.float32)

    degp = _deg_call(dst2, ones_p, zeros_deg)
    w, t1, dinv = _se_call(
        x, U_W, U_b.reshape(1, _D), fc1_W, fc1_b.reshape(1, _D // 4),
        fc2_W, fc2_b.reshape(1, _D), W1, degp[:_N], degp[_NP:_NP + _N])
    agg1 = _agg128(t1, src2, dst2, zeros128)
    (t2,) = _mid_call(agg1[:_N], agg1[_NP:_NP + _N], t1, dinv,
                      b1.reshape(1, _H1), W2)
    agg2 = _agg64(t2, src2, dst2, zeros64)
    (logp,) = _out_call(agg2[:_N], agg2[_NP:_NP + _N], t2, dinv,
                        b2.reshape(1, _H2), fc_W, fc_b.reshape(1, _OUT))
    return (logp, w)


# speedup vs baseline: 1.5474x; 1.0184x over previous
import functools

import jax
import jax.numpy as jnp
from jax import lax
from jax.experimental import pallas as pl
from jax.experimental.pallas import tpu as pltpu
from jax.experimental.pallas import tpu_sc as plsc

_N = 10000
_E = 320000
_D = 128
_H1 = 128
_H2 = 64
_OUT = 40

_NC = 2
_NS = 16
_NW = _NC * _NS

_CH = 125
_ROWS = _E // (_NW * _CH)
_NP = 10240
_RPT = _NP // _NS
_DEGW = 8

_MESH = plsc.VectorSubcoreMesh(
    core_axis_name="c", subcore_axis_name="s", num_cores=_NC, num_subcores=_NS)


def _deg_body(dst2, ones, zeros, out, didx, onesb, acc, gsem):
    cid = lax.axis_index("c")
    sid = lax.axis_index("s")
    wid = cid * _NS + sid
    r0 = sid * _RPT
    pltpu.sync_copy(zeros, acc.at[pl.ds(r0, _RPT)])
    pltpu.sync_copy(ones, onesb)
    pltpu.sync_copy(dst2.at[pl.ds(wid * _ROWS, _ROWS)], didx)
    plsc.subcore_barrier()

    def body(i, carry):
        pltpu.sync_copy(onesb, acc.at[didx.at[i]], add=True)
        return carry

    lax.fori_loop(0, _ROWS, body, 0)
    plsc.subcore_barrier()
    pltpu.sync_copy(acc.at[pl.ds(r0, _RPT)], out.at[pl.ds(cid * _NP + r0, _RPT)])


_deg_call = functools.partial(
    pl.kernel,
    out_type=jax.ShapeDtypeStruct((2 * _NP, _DEGW), jnp.float32),
    mesh=_MESH,
    compiler_params=pltpu.CompilerParams(use_tc_tiling_on_sc=False),
    scratch_types=[
        pltpu.VMEM((_ROWS, _CH), jnp.int32),
        pltpu.VMEM((_CH, _DEGW), jnp.float32),
        pltpu.VMEM_SHARED((_NP, _DEGW), jnp.float32),
        pltpu.SemaphoreType.DMA,
    ],
)(_deg_body)


_Q = 16
_NQ = _ROWS // _Q


def _make_agg(width, tiled=True):
    def _agg_body(table, src2, dst2, zeros, out, sidxb, didxb, rows, acc,
                  gsem0, gsem1, ssem0, ssem1):
        cid = lax.axis_index("c")
        sid = lax.axis_index("s")
        wid = cid * _NS + sid
        r0 = sid * _RPT
        base = wid * _ROWS
        pltpu.sync_copy(zeros, acc.at[pl.ds(r0, _RPT)])
        pltpu.sync_copy(src2.at[pl.ds(base, _Q)], sidxb.at[0])
        pltpu.sync_copy(dst2.at[pl.ds(base, _Q)], didxb.at[0])
        plsc.subcore_barrier()

        def sidx(c):
            return sidxb.at[(c // _Q) % 2, c % _Q]

        def didx(c):
            return didxb.at[(c // _Q) % 2, c % _Q]

        gsem = (gsem0, gsem1)
        ssem = (ssem0, ssem1)

        def wait_g(b):
            pltpu.make_async_copy(
                table.at[sidx(0)], rows.at[b], gsem[b]).wait()

        def wait_s(b):
            pltpu.make_async_copy(
                rows.at[b], acc.at[didx(0)], ssem[b]).wait()

        def issue_g(c, b):
            pltpu.async_copy(table.at[sidx(c)], rows.at[b], gsem[b])

        def issue_s(c, b):
            pltpu.async_copy(rows.at[b], acc.at[didx(c)], ssem[b], add=True)

        issue_g(0, 0)
        wait_g(0)
        issue_s(0, 0)
        issue_g(1, 1)

        def body(j, carry):
            qn = (2 * j + 2) // _Q

            @pl.when((j % (_Q // 2) == _Q // 2 - 1) & (qn < _NQ))
            def _():
                off = base + qn * _Q
                pltpu.sync_copy(src2.at[pl.ds(off, _Q)], sidxb.at[qn % 2])
                pltpu.sync_copy(dst2.at[pl.ds(off, _Q)], didxb.at[qn % 2])

            wait_g(1)
            issue_s(2 * j + 1, 1)
            wait_s(0)
            issue_g(2 * j + 2, 0)
            wait_g(0)
            issue_s(2 * j + 2, 0)
            wait_s(1)

            @pl.when(j < _ROWS // 2 - 2)
            def _():
                issue_g(2 * j + 3, 1)

            return carry

        lax.fori_loop(0, _ROWS // 2 - 1, body, 0)
        issue_g(_ROWS - 1, 1)
        wait_g(1)
        issue_s(_ROWS - 1, 1)
        wait_s(0)
        wait_s(1)
        plsc.subcore_barrier()
        pltpu.sync_copy(acc.at[pl.ds(r0, _RPT)],
                        out.at[pl.ds(cid * _NP + r0, _RPT)])

    return functools.partial(
        pl.kernel,
        out_type=jax.ShapeDtypeStruct((2 * _NP, width), jnp.float32),
        mesh=_MESH,
        compiler_params=None if tiled else pltpu.CompilerParams(
            use_tc_tiling_on_sc=False),
        scratch_types=[
            pltpu.VMEM((2, _Q, _CH), jnp.int32),
            pltpu.VMEM((2, _Q, _CH), jnp.int32),
            pltpu.VMEM((2, _CH, width), jnp.float32),
            pltpu.VMEM_SHARED((_NP, width), jnp.float32),
            pltpu.SemaphoreType.DMA,
            pltpu.SemaphoreType.DMA,
            pltpu.SemaphoreType.DMA,
            pltpu.SemaphoreType.DMA,
        ],
    )(_agg_body)


_agg128 = _make_agg(_D)
_agg64 = _make_agg(_H2, tiled=False)


_BLK = 2000
_GRID = _N // _BLK


def _se_body(x_ref, uw, ub, f1w, f1b, f2w, f2b, w1, d0, d1,
             w_out, t1_out, dinv_out):
    x = x_ref[...]
    h = jnp.dot(x, uw[...], preferred_element_type=jnp.float32) + ub[...]
    t = jnp.maximum(
        jnp.dot(h, f1w[...], preferred_element_type=jnp.float32) + f1b[...], 0.0)
    wgt = jax.nn.sigmoid(
        jnp.dot(t, f2w[...], preferred_element_type=jnp.float32) + f2b[...])
    deg = d0[...][:, 0:1] + d1[...][:, 0:1] + 1.0
    dinv = lax.rsqrt(deg)
    w_out[...] = wgt
    t1_out[...] = jnp.dot(h * wgt * dinv, w1[...],
                          preferred_element_type=jnp.float32)
    dinv_out[...] = dinv


def _mid_body(a0, a1, t1, dinv_ref, b1, w2, t2_out):
    dinv = dinv_ref[...]
    h1 = jnp.maximum(dinv * (a0[...] + a1[...] + t1[...]) + b1[...], 0.0)
    t2_out[...] = dinv * jnp.dot(h1, w2[...],
                                 preferred_element_type=jnp.float32)


def _out_body(a0, a1, t2, dinv_ref, b2, fcw, fcb, lp_out):
    dinv = dinv_ref[...]
    h2 = jnp.maximum(dinv * (a0[...] + a1[...] + t2[...]) + b2[...], 0.0)
    logits = jnp.dot(h2, fcw[...], preferred_element_type=jnp.float32) + fcb[...]
    m = jnp.max(logits, axis=1, keepdims=True)
    s = logits - m
    lp_out[...] = s - jnp.log(jnp.sum(jnp.exp(s), axis=1, keepdims=True))


def _row_spec(w):
    return pl.BlockSpec((_BLK, w), lambda i: (i, 0))


def _full_spec(r, c):
    return pl.BlockSpec((r, c), lambda i: (0, 0))


_se_call = pl.pallas_call(
    _se_body,
    grid=(_GRID,),
    in_specs=[
        _row_spec(_D), _full_spec(_D, _D), _full_spec(1, _D),
        _full_spec(_D, _D // 4), _full_spec(1, _D // 4),
        _full_spec(_D // 4, _D), _full_spec(1, _D),
        _full_spec(_D, _H1), _row_spec(_DEGW), _row_spec(_DEGW),
    ],
    out_specs=[_row_spec(_D), _row_spec(_H1), _row_spec(1)],
    out_shape=[
        jax.ShapeDtypeStruct((_N, _D), jnp.float32),
        jax.ShapeDtypeStruct((_N, _H1), jnp.float32),
        jax.ShapeDtypeStruct((_N, 1), jnp.float32),
    ],
)

_mid_call = pl.pallas_call(
    _mid_body,
    grid=(_GRID,),
    in_specs=[
        _row_spec(_H1), _row_spec(_H1), _row_spec(_H1), _row_spec(1),
        _full_spec(1, _H1), _full_spec(_H1, _H2),
    ],
    out_specs=[_row_spec(_H2)],
    out_shape=[jax.ShapeDtypeStruct((_N, _H2), jnp.float32)],
)

_out_call = pl.pallas_call(
    _out_body,
    grid=(_GRID,),
    in_specs=[
        _row_spec(_H2), _row_spec(_H2), _row_spec(_H2), _row_spec(1),
        _full_spec(1, _H2), _full_spec(_H2, _OUT), _full_spec(1, _OUT),
    ],
    out_specs=[_row_spec(_OUT)],
    out_shape=[jax.ShapeDtypeStruct((_N, _OUT), jnp.float32)],
)


def kernel(x, edge_index, U_W, U_b, fc1_W, fc1_b, fc2_W, fc2_b,
           W1, b1, W2, b2, fc_W, fc_b):
    src2 = edge_index[0].astype(jnp.int32).reshape(_E // _CH, _CH)
    dst2 = edge_index[1].astype(jnp.int32).reshape(_E // _CH, _CH)

    ones_p = jnp.ones((_CH, _DEGW), jnp.float32)
    zeros_deg = jnp.zeros((_RPT, _DEGW), jnp.float32)
    zeros128 = jnp.zeros((_RPT, _D), jnp.float32)
    zeros64 = jnp.zeros((_RPT, _H2), jnp.float32)

    degp = _deg_call(dst2, ones_p, zeros_deg)
    w, t1, dinv = _se_call(
        x, U_W, U_b.reshape(1, _D), fc1_W, fc1_b.reshape(1, _D // 4),
        fc2_W, fc2_b.reshape(1, _D), W1, degp[:_N], degp[_NP:_NP + _N])
    agg1 = _agg128(t1, src2, dst2, zeros128)
    (t2,) = _mid_call(agg1[:_N], agg1[_NP:_NP + _N], t1, dinv,
                      b1.reshape(1, _H1), W2)
    agg2 = _agg64(t2, src2, dst2, zeros64)
    (logp,) = _out_call(agg2[:_N], agg2[_NP:_NP + _N], t2, dinv,
                        b2.reshape(1, _H2), fc_W, fc_b.reshape(1, _OUT))
    return (logp, w)
